# baseline (speedup 1.0000x reference)
import functools

import jax
import jax.numpy as jnp
from jax import lax
from jax.experimental import pallas as pl
from jax.experimental.pallas import tpu as pltpu
from jax.experimental.pallas import tpu_sc as plsc

_NQ = 4096
_K = 8192
_L = 16
_NC = 2
_NS = 16
_NW = _NC * _NS
_SC_Q = 2048
_TC_BLK = 256


def _build_sc_kernel(sq):
    qpw = sq // _NW
    nvec = qpw // _L
    mesh = plsc.VectorSubcoreMesh(core_axis_name="c", subcore_axis_name="s")

    @functools.partial(
        pl.kernel,
        mesh=mesh,
        out_type=[
            jax.ShapeDtypeStruct((sq,), jnp.float32),
            jax.ShapeDtypeStruct((sq,), jnp.float32),
        ],
        scratch_types=[
            pltpu.VMEM((qpw,), jnp.float32),
            pltpu.VMEM((qpw,), jnp.float32),
            pltpu.VMEM((_K,), jnp.float32),
            pltpu.VMEM((_K,), jnp.float32),
            pltpu.VMEM((qpw,), jnp.float32),
            pltpu.VMEM((qpw,), jnp.float32),
            pltpu.VMEM((qpw,), jnp.int32),
            pltpu.VMEM((qpw,), jnp.float32),
            pltpu.VMEM((qpw,), jnp.float32),
            pltpu.SemaphoreType.DMA,
        ],
    )
    def sc_kernel(ax_hbm, ay_hbm, bx_hbm, by_hbm, ox_hbm, oy_hbm,
                  ax_v, ay_v, bx_v, by_v, ox_v, oy_v, idx_v, gx_v, gy_v, sem):
        wid = lax.axis_index("s") * _NC + lax.axis_index("c")
        base = wid * qpw
        pltpu.sync_copy(ax_hbm.at[pl.ds(base, qpw)], ax_v)
        pltpu.sync_copy(ay_hbm.at[pl.ds(base, qpw)], ay_v)
        pltpu.sync_copy(bx_hbm, bx_v)
        pltpu.sync_copy(by_hbm, by_v)

        axs = [ax_v[pl.ds(j * _L, _L)] for j in range(nvec)]
        ays = [ay_v[pl.ds(j * _L, _L)] for j in range(nvec)]

        inf = jnp.full((_L,), jnp.inf, jnp.float32)
        zi = jnp.zeros((_L,), jnp.int32)
        carry0 = tuple([inf] * nvec + [zi] * nvec)

        def body(c, carry):
            ms = list(carry[:nvec])
            kb = list(carry[nvec:])
            kbase = c * _L
            bxc = bx_v[pl.ds(kbase, _L)]
            byc = by_v[pl.ds(kbase, _L)]
            for t in range(_L):
                bxk = bxc[t]
                byk = byc[t]
                kvec = jnp.full((_L,), kbase + t, jnp.int32)
                for j in range(nvec):
                    dx = axs[j] - bxk
                    dy = ays[j] - byk
                    cheb = jnp.maximum(jnp.abs(dx), jnp.abs(dy))
                    pred = cheb < ms[j]
                    ms[j] = jnp.minimum(ms[j], cheb)
                    kb[j] = jnp.where(pred, kvec, kb[j])
            return tuple(ms + kb)

        carry = lax.fori_loop(0, _K // _L, body, carry0)
        for j in range(nvec):
            idx_v[pl.ds(j * _L, _L)] = carry[nvec + j]
        pltpu.async_copy(bx_hbm.at[idx_v], gx_v, sem).wait()
        pltpu.async_copy(by_hbm.at[idx_v], gy_v, sem).wait()
        for j in range(nvec):
            ox_v[pl.ds(j * _L, _L)] = axs[j] - gx_v[pl.ds(j * _L, _L)]
            oy_v[pl.ds(j * _L, _L)] = ays[j] - gy_v[pl.ds(j * _L, _L)]

        pltpu.sync_copy(ox_v, ox_hbm.at[pl.ds(base, qpw)])
        pltpu.sync_copy(oy_v, oy_hbm.at[pl.ds(base, qpw)])

    return sc_kernel


def _tc_body(a_ref, bx_ref, by_ref, iota_ref, o_ref):
    ax = a_ref[:, 0:1]
    ay = a_ref[:, 1:2]
    bx = bx_ref[:, :]
    by = by_ref[:, :]
    iota = iota_ref[:, :]
    cheb = jnp.maximum(jnp.abs(ax - bx), jnp.abs(ay - by))
    m = jnp.min(cheb, axis=1, keepdims=True)
    idxm = jnp.where(cheb == m, iota, jnp.float32(_K))
    idx = jnp.min(idxm, axis=1, keepdims=True)
    onehot = (iota == idx).astype(jnp.bfloat16)
    b2 = jnp.concatenate([bx, by], axis=0)
    hi = b2.astype(jnp.bfloat16)
    r1 = b2 - hi.astype(jnp.float32)
    mid = r1.astype(jnp.bfloat16)
    lo = (r1 - mid.astype(jnp.float32)).astype(jnp.bfloat16)
    b6 = jnp.concatenate([hi, mid, lo], axis=0)
    sel6 = lax.dot_general(onehot, b6, (((1,), (1,)), ((), ())),
                           preferred_element_type=jnp.float32)
    sel = sel6[:, 0:2] + sel6[:, 2:4] + sel6[:, 4:6]
    o_ref[:, :] = jnp.concatenate([ax, ay], axis=1) - sel


def _tc_call(a2, bx, by):
    tq = a2.shape[0]
    grid = (tq // _TC_BLK,)
    aspec = pl.BlockSpec((_TC_BLK, 2), lambda i: (i, 0))
    bspec = pl.BlockSpec((1, _K), lambda i: (0, 0))
    iota_row = jnp.arange(_K, dtype=jnp.float32)[None, :]
    return pl.pallas_call(
        _tc_body,
        grid=grid,
        in_specs=[aspec, bspec, bspec, bspec],
        out_specs=aspec,
        out_shape=jax.ShapeDtypeStruct((tq, 2), jnp.float32),
    )(a2, bx[None, :], by[None, :], iota_row)


def kernel(A, B):
    flat = A.reshape(_NQ, 2)
    bx = B[0]
    by = B[1]
    if _SC_Q == 0:
        out = _tc_call(flat, bx, by)
    elif _SC_Q == _NQ:
        ox, oy = _build_sc_kernel(_NQ)(flat[:, 0], flat[:, 1], bx, by)
        out = jnp.stack([ox, oy], axis=-1)
    else:
        tc_part = _tc_call(flat[_SC_Q:], bx, by)
        ox_sc, oy_sc = _build_sc_kernel(_SC_Q)(
            flat[:_SC_Q, 0], flat[:_SC_Q, 1], bx, by)
        sc_part = jnp.stack([ox_sc, oy_sc], axis=-1)
        out = jnp.concatenate([sc_part, tc_part])
    return out.reshape(A.shape)


# speedup vs baseline: 1.0361x; 1.0361x over previous
import functools

import jax
import jax.numpy as jnp
from jax import lax
from jax.experimental import pallas as pl
from jax.experimental.pallas import tpu as pltpu
from jax.experimental.pallas import tpu_sc as plsc

_NQ = 4096
_K = 8192
_L = 16
_NC = 2
_NS = 16
_NW = _NC * _NS
_SC_Q = 1536
_TC_BLK = 256


def _build_sc_kernel(sq):
    qpw = sq // _NW
    nvec = qpw // _L
    mesh = plsc.VectorSubcoreMesh(core_axis_name="c", subcore_axis_name="s")

    @functools.partial(
        pl.kernel,
        mesh=mesh,
        out_type=[
            jax.ShapeDtypeStruct((sq,), jnp.float32),
            jax.ShapeDtypeStruct((sq,), jnp.float32),
        ],
        scratch_types=[
            pltpu.VMEM((qpw,), jnp.float32),
            pltpu.VMEM((qpw,), jnp.float32),
            pltpu.VMEM((_K,), jnp.float32),
            pltpu.VMEM((_K,), jnp.float32),
            pltpu.VMEM((qpw,), jnp.float32),
            pltpu.VMEM((qpw,), jnp.float32),
            pltpu.VMEM((qpw,), jnp.int32),
            pltpu.VMEM((qpw,), jnp.float32),
            pltpu.VMEM((qpw,), jnp.float32),
            pltpu.SemaphoreType.DMA,
        ],
    )
    def sc_kernel(ax_hbm, ay_hbm, bx_hbm, by_hbm, ox_hbm, oy_hbm,
                  ax_v, ay_v, bx_v, by_v, ox_v, oy_v, idx_v, gx_v, gy_v, sem):
        wid = lax.axis_index("s") * _NC + lax.axis_index("c")
        base = wid * qpw
        pltpu.sync_copy(ax_hbm.at[pl.ds(base, qpw)], ax_v)
        pltpu.sync_copy(ay_hbm.at[pl.ds(base, qpw)], ay_v)
        pltpu.sync_copy(bx_hbm, bx_v)
        pltpu.sync_copy(by_hbm, by_v)

        axs = [ax_v[pl.ds(j * _L, _L)] for j in range(nvec)]
        ays = [ay_v[pl.ds(j * _L, _L)] for j in range(nvec)]

        inf = jnp.full((_L,), jnp.inf, jnp.float32)
        zi = jnp.zeros((_L,), jnp.int32)
        carry0 = tuple([inf] * nvec + [zi] * nvec)

        def body(c, carry):
            ms = list(carry[:nvec])
            kb = list(carry[nvec:])
            kbase = c * _L
            bxc = bx_v[pl.ds(kbase, _L)]
            byc = by_v[pl.ds(kbase, _L)]
            for t in range(_L):
                bxk = bxc[t]
                byk = byc[t]
                kvec = jnp.full((_L,), kbase + t, jnp.int32)
                for j in range(nvec):
                    dx = axs[j] - bxk
                    dy = ays[j] - byk
                    cheb = jnp.maximum(jnp.abs(dx), jnp.abs(dy))
                    pred = cheb < ms[j]
                    ms[j] = jnp.minimum(ms[j], cheb)
                    kb[j] = jnp.where(pred, kvec, kb[j])
            return tuple(ms + kb)

        carry = lax.fori_loop(0, _K // _L, body, carry0)
        for j in range(nvec):
            idx_v[pl.ds(j * _L, _L)] = carry[nvec + j]
        pltpu.async_copy(bx_hbm.at[idx_v], gx_v, sem).wait()
        pltpu.async_copy(by_hbm.at[idx_v], gy_v, sem).wait()
        for j in range(nvec):
            ox_v[pl.ds(j * _L, _L)] = axs[j] - gx_v[pl.ds(j * _L, _L)]
            oy_v[pl.ds(j * _L, _L)] = ays[j] - gy_v[pl.ds(j * _L, _L)]

        pltpu.sync_copy(ox_v, ox_hbm.at[pl.ds(base, qpw)])
        pltpu.sync_copy(oy_v, oy_hbm.at[pl.ds(base, qpw)])

    return sc_kernel


def _tc_body(a_ref, bx_ref, by_ref, iota_ref, o_ref):
    ax = a_ref[:, 0:1]
    ay = a_ref[:, 1:2]
    bx = bx_ref[:, :]
    by = by_ref[:, :]
    iota = iota_ref[:, :]
    cheb = jnp.maximum(jnp.abs(ax - bx), jnp.abs(ay - by))
    m = jnp.min(cheb, axis=1, keepdims=True)
    idxm = jnp.where(cheb == m, iota, jnp.float32(_K))
    idx = jnp.min(idxm, axis=1, keepdims=True)
    onehot = (iota == idx).astype(jnp.bfloat16)
    b2 = jnp.concatenate([bx, by], axis=0)
    hi = b2.astype(jnp.bfloat16)
    r1 = b2 - hi.astype(jnp.float32)
    mid = r1.astype(jnp.bfloat16)
    lo = (r1 - mid.astype(jnp.float32)).astype(jnp.bfloat16)
    b6 = jnp.concatenate([hi, mid, lo], axis=0)
    sel6 = lax.dot_general(onehot, b6, (((1,), (1,)), ((), ())),
                           preferred_element_type=jnp.float32)
    sel = sel6[:, 0:2] + sel6[:, 2:4] + sel6[:, 4:6]
    o_ref[:, :] = jnp.concatenate([ax, ay], axis=1) - sel


def _tc_call(a2, bx, by):
    tq = a2.shape[0]
    grid = (tq // _TC_BLK,)
    aspec = pl.BlockSpec((_TC_BLK, 2), lambda i: (i, 0))
    bspec = pl.BlockSpec((1, _K), lambda i: (0, 0))
    iota_row = jnp.arange(_K, dtype=jnp.float32)[None, :]
    return pl.pallas_call(
        _tc_body,
        grid=grid,
        in_specs=[aspec, bspec, bspec, bspec],
        out_specs=aspec,
        out_shape=jax.ShapeDtypeStruct((tq, 2), jnp.float32),
    )(a2, bx[None, :], by[None, :], iota_row)


def _asm_body(ox_ref, oy_ref, tcp_ref, o_ref):
    o_ref[0:_SC_Q, 0:1] = ox_ref[:, :]
    o_ref[0:_SC_Q, 1:2] = oy_ref[:, :]
    o_ref[_SC_Q:_NQ, :] = tcp_ref[:, :]


def kernel(A, B):
    flat = A.reshape(_NQ, 2)
    bx = B[0]
    by = B[1]
    if _SC_Q == 0:
        out = _tc_call(flat, bx, by)
    elif _SC_Q == _NQ:
        ox, oy = _build_sc_kernel(_NQ)(flat[:, 0], flat[:, 1], bx, by)
        out = jnp.stack([ox, oy], axis=-1)
    else:
        tc_part = _tc_call(flat[_SC_Q:], bx, by)
        ox_sc, oy_sc = _build_sc_kernel(_SC_Q)(
            flat[:_SC_Q, 0], flat[:_SC_Q, 1], bx, by)
        out = pl.pallas_call(
            _asm_body,
            out_shape=jax.ShapeDtypeStruct((_NQ, 2), jnp.float32),
        )(ox_sc[:, None], oy_sc[:, None], tc_part)
    return out.reshape(A.shape)


# speedup vs baseline: 1.1678x; 1.1271x over previous
import functools

import jax
import jax.numpy as jnp
from jax import lax
from jax.experimental import pallas as pl
from jax.experimental.pallas import tpu as pltpu
from jax.experimental.pallas import tpu_sc as plsc

_NQ = 4096
_K = 8192
_L = 16
_NC = 2
_NS = 16
_NW = _NC * _NS
_SC_Q = 1536
_TC_BLK = 512


def _build_sc_kernel(sq):
    qpw = sq // _NW
    nvec = qpw // _L
    mesh = plsc.VectorSubcoreMesh(core_axis_name="c", subcore_axis_name="s")

    @functools.partial(
        pl.kernel,
        mesh=mesh,
        out_type=[
            jax.ShapeDtypeStruct((sq,), jnp.float32),
            jax.ShapeDtypeStruct((sq,), jnp.float32),
        ],
        scratch_types=[
            pltpu.VMEM((qpw,), jnp.float32),
            pltpu.VMEM((qpw,), jnp.float32),
            pltpu.VMEM((_K,), jnp.float32),
            pltpu.VMEM((_K,), jnp.float32),
            pltpu.VMEM((qpw,), jnp.float32),
            pltpu.VMEM((qpw,), jnp.float32),
            pltpu.VMEM((qpw,), jnp.int32),
            pltpu.VMEM((qpw,), jnp.float32),
            pltpu.VMEM((qpw,), jnp.float32),
            pltpu.SemaphoreType.DMA,
        ],
    )
    def sc_kernel(ax_hbm, ay_hbm, bx_hbm, by_hbm, ox_hbm, oy_hbm,
                  ax_v, ay_v, bx_v, by_v, ox_v, oy_v, idx_v, gx_v, gy_v, sem):
        wid = lax.axis_index("s") * _NC + lax.axis_index("c")
        base = wid * qpw
        pltpu.sync_copy(ax_hbm.at[pl.ds(base, qpw)], ax_v)
        pltpu.sync_copy(ay_hbm.at[pl.ds(base, qpw)], ay_v)
        pltpu.sync_copy(bx_hbm, bx_v)
        pltpu.sync_copy(by_hbm, by_v)

        axs = [ax_v[pl.ds(j * _L, _L)] for j in range(nvec)]
        ays = [ay_v[pl.ds(j * _L, _L)] for j in range(nvec)]

        inf = jnp.full((_L,), jnp.inf, jnp.float32)
        zi = jnp.zeros((_L,), jnp.int32)
        carry0 = tuple([inf] * nvec + [zi] * nvec)

        def body(c, carry):
            ms = list(carry[:nvec])
            kb = list(carry[nvec:])
            kbase = c * _L
            bxc = bx_v[pl.ds(kbase, _L)]
            byc = by_v[pl.ds(kbase, _L)]
            for t in range(_L):
                bxk = bxc[t]
                byk = byc[t]
                kvec = jnp.full((_L,), kbase + t, jnp.int32)
                for j in range(nvec):
                    dx = axs[j] - bxk
                    dy = ays[j] - byk
                    cheb = jnp.maximum(jnp.abs(dx), jnp.abs(dy))
                    pred = cheb < ms[j]
                    ms[j] = jnp.minimum(ms[j], cheb)
                    kb[j] = jnp.where(pred, kvec, kb[j])
            return tuple(ms + kb)

        carry = lax.fori_loop(0, _K // _L, body, carry0)
        for j in range(nvec):
            idx_v[pl.ds(j * _L, _L)] = carry[nvec + j]
        pltpu.async_copy(bx_hbm.at[idx_v], gx_v, sem).wait()
        pltpu.async_copy(by_hbm.at[idx_v], gy_v, sem).wait()
        for j in range(nvec):
            ox_v[pl.ds(j * _L, _L)] = axs[j] - gx_v[pl.ds(j * _L, _L)]
            oy_v[pl.ds(j * _L, _L)] = ays[j] - gy_v[pl.ds(j * _L, _L)]

        pltpu.sync_copy(ox_v, ox_hbm.at[pl.ds(base, qpw)])
        pltpu.sync_copy(oy_v, oy_hbm.at[pl.ds(base, qpw)])

    return sc_kernel


def _tc_body(a_ref, bx_ref, by_ref, iota_ref, o_ref):
    ax = a_ref[:, 0:1]
    ay = a_ref[:, 1:2]
    bx = bx_ref[:, :]
    by = by_ref[:, :]
    iota = iota_ref[:, :]
    cheb = jnp.maximum(jnp.abs(ax - bx), jnp.abs(ay - by))
    m = jnp.min(cheb, axis=1, keepdims=True)
    idxm = jnp.where(cheb == m, iota, jnp.float32(_K))
    idx = jnp.min(idxm, axis=1, keepdims=True)
    onehot = (iota == idx).astype(jnp.bfloat16)
    b2 = jnp.concatenate([bx, by], axis=0)
    hi = b2.astype(jnp.bfloat16)
    r1 = b2 - hi.astype(jnp.float32)
    mid = r1.astype(jnp.bfloat16)
    lo = (r1 - mid.astype(jnp.float32)).astype(jnp.bfloat16)
    b6 = jnp.concatenate([hi, mid, lo], axis=0)
    sel6 = lax.dot_general(onehot, b6, (((1,), (1,)), ((), ())),
                           preferred_element_type=jnp.float32)
    sel = sel6[:, 0:2] + sel6[:, 2:4] + sel6[:, 4:6]
    o_ref[:, :] = jnp.concatenate([ax, ay], axis=1) - sel


def _tc_call(a2, bx, by):
    tq = a2.shape[0]
    grid = (tq // _TC_BLK,)
    aspec = pl.BlockSpec((_TC_BLK, 2), lambda i: (i, 0))
    bspec = pl.BlockSpec((1, _K), lambda i: (0, 0))
    iota_row = jnp.arange(_K, dtype=jnp.float32)[None, :]
    return pl.pallas_call(
        _tc_body,
        grid=grid,
        in_specs=[aspec, bspec, bspec, bspec],
        out_specs=aspec,
        out_shape=jax.ShapeDtypeStruct((tq, 2), jnp.float32),
    )(a2, bx[None, :], by[None, :], iota_row)


def kernel(A, B):
    flat = A.reshape(_NQ, 2)
    bx = B[0]
    by = B[1]
    if _SC_Q == 0:
        out = _tc_call(flat, bx, by)
    elif _SC_Q == _NQ:
        ox, oy = _build_sc_kernel(_NQ)(flat[:, 0], flat[:, 1], bx, by)
        out = jnp.stack([ox, oy], axis=-1)
    else:
        tc_part = _tc_call(flat[_SC_Q:], bx, by)
        ox_sc, oy_sc = _build_sc_kernel(_SC_Q)(
            flat[:_SC_Q, 0], flat[:_SC_Q, 1], bx, by)
        sc_part = jnp.stack([ox_sc, oy_sc], axis=-1)
        out = jnp.concatenate([sc_part, tc_part])
    return out.reshape(A.shape)


# speedup vs baseline: 1.1689x; 1.0009x over previous
import functools

import jax
import jax.numpy as jnp
from jax import lax
from jax.experimental import pallas as pl
from jax.experimental.pallas import tpu as pltpu
from jax.experimental.pallas import tpu_sc as plsc

_NQ = 4096
_K = 8192
_L = 16
_NC = 2
_NS = 16
_NW = _NC * _NS
_SC_Q = 1536
_TC_BLK = 256


def _build_sc_kernel(sq):
    qpw = sq // _NW
    nvec = qpw // _L
    mesh = plsc.VectorSubcoreMesh(core_axis_name="c", subcore_axis_name="s")

    @functools.partial(
        pl.kernel,
        mesh=mesh,
        out_type=[
            jax.ShapeDtypeStruct((sq,), jnp.float32),
            jax.ShapeDtypeStruct((sq,), jnp.float32),
        ],
        scratch_types=[
            pltpu.VMEM((qpw,), jnp.float32),
            pltpu.VMEM((qpw,), jnp.float32),
            pltpu.VMEM((_K,), jnp.float32),
            pltpu.VMEM((_K,), jnp.float32),
            pltpu.VMEM((qpw,), jnp.float32),
            pltpu.VMEM((qpw,), jnp.float32),
            pltpu.VMEM((qpw,), jnp.int32),
            pltpu.VMEM((qpw,), jnp.float32),
            pltpu.VMEM((qpw,), jnp.float32),
            pltpu.SemaphoreType.DMA,
        ],
    )
    def sc_kernel(ax_hbm, ay_hbm, bx_hbm, by_hbm, ox_hbm, oy_hbm,
                  ax_v, ay_v, bx_v, by_v, ox_v, oy_v, idx_v, gx_v, gy_v, sem):
        wid = lax.axis_index("s") * _NC + lax.axis_index("c")
        base = wid * qpw
        pltpu.sync_copy(ax_hbm.at[pl.ds(base, qpw)], ax_v)
        pltpu.sync_copy(ay_hbm.at[pl.ds(base, qpw)], ay_v)
        pltpu.sync_copy(bx_hbm, bx_v)
        pltpu.sync_copy(by_hbm, by_v)

        axs = [ax_v[pl.ds(j * _L, _L)] for j in range(nvec)]
        ays = [ay_v[pl.ds(j * _L, _L)] for j in range(nvec)]

        inf = jnp.full((_L,), jnp.inf, jnp.float32)
        zi = jnp.zeros((_L,), jnp.int32)
        carry0 = tuple([inf] * nvec + [zi] * nvec)

        def body(c, carry):
            ms = list(carry[:nvec])
            kb = list(carry[nvec:])
            kbase = c * _L
            bxc = bx_v[pl.ds(kbase, _L)]
            byc = by_v[pl.ds(kbase, _L)]
            for t in range(_L):
                bxk = bxc[t]
                byk = byc[t]
                kvec = jnp.full((_L,), kbase + t, jnp.int32)
                for j in range(nvec):
                    dx = axs[j] - bxk
                    dy = ays[j] - byk
                    cheb = jnp.maximum(jnp.abs(dx), jnp.abs(dy))
                    pred = cheb < ms[j]
                    ms[j] = jnp.minimum(ms[j], cheb)
                    kb[j] = jnp.where(pred, kvec, kb[j])
            return tuple(ms + kb)

        carry = lax.fori_loop(0, _K // _L, body, carry0)
        for j in range(nvec):
            idx_v[pl.ds(j * _L, _L)] = carry[nvec + j]
        pltpu.async_copy(bx_hbm.at[idx_v], gx_v, sem).wait()
        pltpu.async_copy(by_hbm.at[idx_v], gy_v, sem).wait()
        for j in range(nvec):
            ox_v[pl.ds(j * _L, _L)] = axs[j] - gx_v[pl.ds(j * _L, _L)]
            oy_v[pl.ds(j * _L, _L)] = ays[j] - gy_v[pl.ds(j * _L, _L)]

        pltpu.sync_copy(ox_v, ox_hbm.at[pl.ds(base, qpw)])
        pltpu.sync_copy(oy_v, oy_hbm.at[pl.ds(base, qpw)])

    return sc_kernel


def _tc_body(a_ref, bx_ref, by_ref, iota_ref, o_ref):
    ax = a_ref[:, 0:1]
    ay = a_ref[:, 1:2]
    bx = bx_ref[:, :]
    by = by_ref[:, :]
    iota = iota_ref[:, :]
    cheb = jnp.maximum(jnp.abs(ax - bx), jnp.abs(ay - by))
    m = jnp.min(cheb, axis=1, keepdims=True)
    idxm = jnp.where(cheb == m, iota, jnp.float32(_K))
    idx = jnp.min(idxm, axis=1, keepdims=True)
    onehot = (iota == idx).astype(jnp.bfloat16)
    b2 = jnp.concatenate([bx, by], axis=0)
    hi = b2.astype(jnp.bfloat16)
    r1 = b2 - hi.astype(jnp.float32)
    mid = r1.astype(jnp.bfloat16)
    lo = (r1 - mid.astype(jnp.float32)).astype(jnp.bfloat16)
    b6 = jnp.concatenate([hi, mid, lo], axis=0)
    sel6 = lax.dot_general(onehot, b6, (((1,), (1,)), ((), ())),
                           preferred_element_type=jnp.float32)
    sel = sel6[:, 0:2] + sel6[:, 2:4] + sel6[:, 4:6]
    o_ref[:, :] = jnp.concatenate([ax, ay], axis=1) - sel


def _tc_call(a2, bx, by):
    tq = a2.shape[0]
    grid = (tq // _TC_BLK,)
    aspec = pl.BlockSpec((_TC_BLK, 2), lambda i: (i, 0))
    bspec = pl.BlockSpec((1, _K), lambda i: (0, 0))
    iota_row = jnp.arange(_K, dtype=jnp.float32)[None, :]
    return pl.pallas_call(
        _tc_body,
        grid=grid,
        in_specs=[aspec, bspec, bspec, bspec],
        out_specs=aspec,
        out_shape=jax.ShapeDtypeStruct((tq, 2), jnp.float32),
    )(a2, bx[None, :], by[None, :], iota_row)


def kernel(A, B):
    flat = A.reshape(_NQ, 2)
    bx = B[0]
    by = B[1]
    if _SC_Q == 0:
        out = _tc_call(flat, bx, by)
    elif _SC_Q == _NQ:
        ox, oy = _build_sc_kernel(_NQ)(flat[:, 0], flat[:, 1], bx, by)
        out = jnp.stack([ox, oy], axis=-1)
    else:
        ox_sc, oy_sc = _build_sc_kernel(_SC_Q)(
            flat[:_SC_Q, 0], flat[:_SC_Q, 1], bx, by)
        tc_part = _tc_call(flat[_SC_Q:], bx, by)
        sc_part = jnp.stack([ox_sc, oy_sc], axis=-1)
        out = jnp.concatenate([sc_part, tc_part])
    return out.reshape(A.shape)


# speedup vs baseline: 1.1814x; 1.0107x over previous
import functools

import jax
import jax.numpy as jnp
from jax import lax
from jax.experimental import pallas as pl
from jax.experimental.pallas import tpu as pltpu
from jax.experimental.pallas import tpu_sc as plsc

_NQ = 4096
_K = 8192
_L = 16
_NC = 2
_NS = 16
_NW = _NC * _NS
_SC_Q = 1536
_TC_BLK = 256


def _build_sc_kernel(sq):
    qpw = sq // _NW
    nvec = qpw // _L
    mesh = plsc.VectorSubcoreMesh(core_axis_name="c", subcore_axis_name="s")

    @functools.partial(
        pl.kernel,
        mesh=mesh,
        out_type=[
            jax.ShapeDtypeStruct((sq,), jnp.float32),
            jax.ShapeDtypeStruct((sq,), jnp.float32),
        ],
        scratch_types=[
            pltpu.VMEM((qpw,), jnp.float32),
            pltpu.VMEM((qpw,), jnp.float32),
            pltpu.VMEM((_K,), jnp.float32),
            pltpu.VMEM((_K,), jnp.float32),
            pltpu.VMEM((qpw,), jnp.float32),
            pltpu.VMEM((qpw,), jnp.float32),
            pltpu.VMEM((qpw,), jnp.int32),
            pltpu.VMEM((qpw,), jnp.float32),
            pltpu.VMEM((qpw,), jnp.float32),
            pltpu.SemaphoreType.DMA,
        ],
    )
    def sc_kernel(ax_hbm, ay_hbm, bx_hbm, by_hbm, ox_hbm, oy_hbm,
                  ax_v, ay_v, bx_v, by_v, ox_v, oy_v, idx_v, gx_v, gy_v, sem):
        wid = lax.axis_index("s") * _NC + lax.axis_index("c")
        base = wid * qpw
        pltpu.sync_copy(ax_hbm.at[pl.ds(base, qpw)], ax_v)
        pltpu.sync_copy(ay_hbm.at[pl.ds(base, qpw)], ay_v)
        pltpu.sync_copy(bx_hbm, bx_v)
        pltpu.sync_copy(by_hbm, by_v)

        axs = [ax_v[pl.ds(j * _L, _L)] for j in range(nvec)]
        ays = [ay_v[pl.ds(j * _L, _L)] for j in range(nvec)]

        inf = jnp.full((_L,), jnp.inf, jnp.float32)
        zi = jnp.zeros((_L,), jnp.int32)
        carry0 = tuple([inf] * nvec + [zi] * nvec)

        def body(c, carry):
            ms = list(carry[:nvec])
            kb = list(carry[nvec:])
            kbase = c * _L
            bxc = bx_v[pl.ds(kbase, _L)]
            byc = by_v[pl.ds(kbase, _L)]
            for t in range(_L):
                bxk = bxc[t]
                byk = byc[t]
                kvec = jnp.full((_L,), kbase + t, jnp.int32)
                for j in range(nvec):
                    dx = axs[j] - bxk
                    dy = ays[j] - byk
                    cheb = jnp.maximum(jnp.abs(dx), jnp.abs(dy))
                    pred = cheb < ms[j]
                    ms[j] = jnp.minimum(ms[j], cheb)
                    kb[j] = jnp.where(pred, kvec, kb[j])
            return tuple(ms + kb)

        carry = lax.fori_loop(0, _K // _L, body, carry0, unroll=2)
        for j in range(nvec):
            idx_v[pl.ds(j * _L, _L)] = carry[nvec + j]
        pltpu.async_copy(bx_hbm.at[idx_v], gx_v, sem).wait()
        pltpu.async_copy(by_hbm.at[idx_v], gy_v, sem).wait()
        for j in range(nvec):
            ox_v[pl.ds(j * _L, _L)] = axs[j] - gx_v[pl.ds(j * _L, _L)]
            oy_v[pl.ds(j * _L, _L)] = ays[j] - gy_v[pl.ds(j * _L, _L)]

        pltpu.sync_copy(ox_v, ox_hbm.at[pl.ds(base, qpw)])
        pltpu.sync_copy(oy_v, oy_hbm.at[pl.ds(base, qpw)])

    return sc_kernel


def _tc_body(a_ref, bx_ref, by_ref, iota_ref, o_ref):
    ax = a_ref[:, 0:1]
    ay = a_ref[:, 1:2]
    bx = bx_ref[:, :]
    by = by_ref[:, :]
    iota = iota_ref[:, :]
    cheb = jnp.maximum(jnp.abs(ax - bx), jnp.abs(ay - by))
    m = jnp.min(cheb, axis=1, keepdims=True)
    idxm = jnp.where(cheb == m, iota, jnp.float32(_K))
    idx = jnp.min(idxm, axis=1, keepdims=True)
    onehot = (iota == idx).astype(jnp.bfloat16)
    b2 = jnp.concatenate([bx, by], axis=0)
    hi = b2.astype(jnp.bfloat16)
    r1 = b2 - hi.astype(jnp.float32)
    mid = r1.astype(jnp.bfloat16)
    lo = (r1 - mid.astype(jnp.float32)).astype(jnp.bfloat16)
    b6 = jnp.concatenate([hi, mid, lo], axis=0)
    sel6 = lax.dot_general(onehot, b6, (((1,), (1,)), ((), ())),
                           preferred_element_type=jnp.float32)
    sel = sel6[:, 0:2] + sel6[:, 2:4] + sel6[:, 4:6]
    o_ref[:, :] = jnp.concatenate([ax, ay], axis=1) - sel


def _tc_call(a2, bx, by):
    tq = a2.shape[0]
    grid = (tq // _TC_BLK,)
    aspec = pl.BlockSpec((_TC_BLK, 2), lambda i: (i, 0))
    bspec = pl.BlockSpec((1, _K), lambda i: (0, 0))
    iota_row = jnp.arange(_K, dtype=jnp.float32)[None, :]
    return pl.pallas_call(
        _tc_body,
        grid=grid,
        in_specs=[aspec, bspec, bspec, bspec],
        out_specs=aspec,
        out_shape=jax.ShapeDtypeStruct((tq, 2), jnp.float32),
    )(a2, bx[None, :], by[None, :], iota_row)


def kernel(A, B):
    flat = A.reshape(_NQ, 2)
    bx = B[0]
    by = B[1]
    if _SC_Q == 0:
        out = _tc_call(flat, bx, by)
    elif _SC_Q == _NQ:
        ox, oy = _build_sc_kernel(_NQ)(flat[:, 0], flat[:, 1], bx, by)
        out = jnp.stack([ox, oy], axis=-1)
    else:
        ox_sc, oy_sc = _build_sc_kernel(_SC_Q)(
            flat[:_SC_Q, 0], flat[:_SC_Q, 1], bx, by)
        tc_part = _tc_call(flat[_SC_Q:], bx, by)
        sc_part = jnp.stack([ox_sc, oy_sc], axis=-1)
        out = jnp.concatenate([sc_part, tc_part])
    return out.reshape(A.shape)


# speedup vs baseline: 1.1893x; 1.0066x over previous
import functools

import jax
import jax.numpy as jnp
from jax import lax
from jax.experimental import pallas as pl
from jax.experimental.pallas import tpu as pltpu
from jax.experimental.pallas import tpu_sc as plsc

_NQ = 4096
_K = 8192
_L = 16
_NC = 2
_NS = 16
_NW = _NC * _NS
_SC_Q = 1536
_TC_BLK = 256


def _build_sc_kernel(sq):
    qpw = sq // _NW
    nvec = qpw // _L
    mesh = plsc.VectorSubcoreMesh(core_axis_name="c", subcore_axis_name="s")

    @functools.partial(
        pl.kernel,
        mesh=mesh,
        out_type=[
            jax.ShapeDtypeStruct((sq,), jnp.float32),
            jax.ShapeDtypeStruct((sq,), jnp.float32),
        ],
        scratch_types=[
            pltpu.VMEM((qpw,), jnp.float32),
            pltpu.VMEM((qpw,), jnp.float32),
            pltpu.VMEM((_K,), jnp.float32),
            pltpu.VMEM((_K,), jnp.float32),
            pltpu.VMEM((qpw,), jnp.float32),
            pltpu.VMEM((qpw,), jnp.float32),
            pltpu.VMEM((qpw,), jnp.int32),
            pltpu.VMEM((qpw,), jnp.float32),
            pltpu.VMEM((qpw,), jnp.float32),
            pltpu.SemaphoreType.DMA,
        ],
    )
    def sc_kernel(ax_hbm, ay_hbm, bx_hbm, by_hbm, ox_hbm, oy_hbm,
                  ax_v, ay_v, bx_v, by_v, ox_v, oy_v, idx_v, gx_v, gy_v, sem):
        wid = lax.axis_index("s") * _NC + lax.axis_index("c")
        base = wid * qpw
        pltpu.sync_copy(ax_hbm.at[pl.ds(base, qpw)], ax_v)
        pltpu.sync_copy(ay_hbm.at[pl.ds(base, qpw)], ay_v)
        pltpu.sync_copy(bx_hbm, bx_v)
        pltpu.sync_copy(by_hbm, by_v)

        axs = [ax_v[pl.ds(j * _L, _L)] for j in range(nvec)]
        ays = [ay_v[pl.ds(j * _L, _L)] for j in range(nvec)]

        inf = jnp.full((_L,), jnp.inf, jnp.float32)
        zi = jnp.zeros((_L,), jnp.int32)
        carry0 = tuple([inf] * nvec + [zi] * nvec)

        def body(c, carry):
            ms = list(carry[:nvec])
            kb = list(carry[nvec:])
            kbase = c * _L
            bxc = bx_v[pl.ds(kbase, _L)]
            byc = by_v[pl.ds(kbase, _L)]
            for t in range(_L):
                bxk = bxc[t]
                byk = byc[t]
                kvec = jnp.full((_L,), kbase + t, jnp.int32)
                for j in range(nvec):
                    dx = axs[j] - bxk
                    dy = ays[j] - byk
                    cheb = jnp.maximum(jnp.abs(dx), jnp.abs(dy))
                    pred = cheb < ms[j]
                    ms[j] = jnp.minimum(ms[j], cheb)
                    kb[j] = jnp.where(pred, kvec, kb[j])
            return tuple(ms + kb)

        carry = lax.fori_loop(0, _K // _L, body, carry0, unroll=4)
        for j in range(nvec):
            idx_v[pl.ds(j * _L, _L)] = carry[nvec + j]
        pltpu.async_copy(bx_hbm.at[idx_v], gx_v, sem).wait()
        pltpu.async_copy(by_hbm.at[idx_v], gy_v, sem).wait()
        for j in range(nvec):
            ox_v[pl.ds(j * _L, _L)] = axs[j] - gx_v[pl.ds(j * _L, _L)]
            oy_v[pl.ds(j * _L, _L)] = ays[j] - gy_v[pl.ds(j * _L, _L)]

        pltpu.sync_copy(ox_v, ox_hbm.at[pl.ds(base, qpw)])
        pltpu.sync_copy(oy_v, oy_hbm.at[pl.ds(base, qpw)])

    return sc_kernel


def _tc_body(a_ref, bx_ref, by_ref, iota_ref, o_ref):
    ax = a_ref[:, 0:1]
    ay = a_ref[:, 1:2]
    bx = bx_ref[:, :]
    by = by_ref[:, :]
    iota = iota_ref[:, :]
    cheb = jnp.maximum(jnp.abs(ax - bx), jnp.abs(ay - by))
    m = jnp.min(cheb, axis=1, keepdims=True)
    idxm = jnp.where(cheb == m, iota, jnp.float32(_K))
    idx = jnp.min(idxm, axis=1, keepdims=True)
    onehot = (iota == idx).astype(jnp.bfloat16)
    b2 = jnp.concatenate([bx, by], axis=0)
    hi = b2.astype(jnp.bfloat16)
    r1 = b2 - hi.astype(jnp.float32)
    mid = r1.astype(jnp.bfloat16)
    lo = (r1 - mid.astype(jnp.float32)).astype(jnp.bfloat16)
    b6 = jnp.concatenate([hi, mid, lo], axis=0)
    sel6 = lax.dot_general(onehot, b6, (((1,), (1,)), ((), ())),
                           preferred_element_type=jnp.float32)
    sel = sel6[:, 0:2] + sel6[:, 2:4] + sel6[:, 4:6]
    o_ref[:, :] = jnp.concatenate([ax, ay], axis=1) - sel


def _tc_call(a2, bx, by):
    tq = a2.shape[0]
    grid = (tq // _TC_BLK,)
    aspec = pl.BlockSpec((_TC_BLK, 2), lambda i: (i, 0))
    bspec = pl.BlockSpec((1, _K), lambda i: (0, 0))
    iota_row = jnp.arange(_K, dtype=jnp.float32)[None, :]
    return pl.pallas_call(
        _tc_body,
        grid=grid,
        in_specs=[aspec, bspec, bspec, bspec],
        out_specs=aspec,
        out_shape=jax.ShapeDtypeStruct((tq, 2), jnp.float32),
    )(a2, bx[None, :], by[None, :], iota_row)


def kernel(A, B):
    flat = A.reshape(_NQ, 2)
    bx = B[0]
    by = B[1]
    if _SC_Q == 0:
        out = _tc_call(flat, bx, by)
    elif _SC_Q == _NQ:
        ox, oy = _build_sc_kernel(_NQ)(flat[:, 0], flat[:, 1], bx, by)
        out = jnp.stack([ox, oy], axis=-1)
    else:
        ox_sc, oy_sc = _build_sc_kernel(_SC_Q)(
            flat[:_SC_Q, 0], flat[:_SC_Q, 1], bx, by)
        tc_part = _tc_call(flat[_SC_Q:], bx, by)
        sc_part = jnp.stack([ox_sc, oy_sc], axis=-1)
        out = jnp.concatenate([sc_part, tc_part])
    return out.reshape(A.shape)
